# baseline (speedup 1.0000x reference)
import functools

import jax
import jax.numpy as jnp
from jax import lax
from jax.experimental import pallas as pl
from jax.experimental.pallas import tpu as pltpu
from jax.experimental.pallas import tpu_sc as plsc

F32 = jnp.float32
_HIGH = lax.Precision.HIGHEST

NW = 32
GK = 80


def _tc1_body(x_ref, g_ref, b_ref, wl_ref, bl_ref, wr_ref, br_ref,
              xl_ref, xr_ref):
    xv = x_ref[...]
    mu = jnp.mean(xv, axis=-1, keepdims=True)
    var = jnp.mean((xv - mu) * (xv - mu), axis=-1, keepdims=True)
    h = (xv - mu) * lax.rsqrt(var + 1e-5) * g_ref[...] + b_ref[...]
    h = jnp.maximum(h, 0.0)
    xl_ref[...] = jnp.dot(h, wl_ref[...], precision=_HIGH,
                          preferred_element_type=F32) + bl_ref[...]
    xr_ref[...] = jnp.dot(h, wr_ref[...], precision=_HIGH,
                          preferred_element_type=F32) + br_ref[...]


def _tc1(x, ln1_g, ln1_b, Wl, bl, Wr, br):
    n, d = x.shape
    nb = 2000
    grid = (n // nb,)
    row = pl.BlockSpec((nb, d), lambda i: (i, 0))
    full = pl.BlockSpec((1, d), lambda i: (0, 0))
    w = pl.BlockSpec((d, d), lambda i: (0, 0))
    return pl.pallas_call(
        _tc1_body,
        grid=grid,
        in_specs=[row, full, full, w, full, w, full],
        out_specs=[row, row],
        out_shape=[jax.ShapeDtypeStruct((n, d), F32)] * 2,
    )(x, ln1_g.reshape(1, d), ln1_b.reshape(1, d), Wl, bl.reshape(1, d),
      Wr, br.reshape(1, d))


def _tc2_body(gxl_ref, gxr_ref, ea_ref, we_ref, att_ref, out_ref, ext_ref):
    d = 128
    ea = jnp.dot(ea_ref[...], we_ref[...], precision=_HIGH,
                 preferred_element_type=F32)
    e = gxl_ref[...] + gxr_ref[...] + ea
    e = jnp.where(e >= 0, e, 0.2 * e)
    p = e * att_ref[...]
    r128 = lax.broadcasted_iota(jnp.int32, (d, 4), 0)
    c4 = lax.broadcasted_iota(jnp.int32, (d, 4), 1)
    sel = (r128 // 32 == c4).astype(F32)
    logits = jnp.dot(p, sel, precision=_HIGH, preferred_element_type=F32)
    ex = jnp.exp(logits)
    r4 = lax.broadcasted_iota(jnp.int32, (4, d), 0)
    c128 = lax.broadcasted_iota(jnp.int32, (4, d), 1)
    bsel = (c128 // 32 == r4).astype(F32)
    exb = jnp.dot(ex, bsel, precision=_HIGH, preferred_element_type=F32)
    out_ref[...] = gxl_ref[...] * exb
    lt = lax.dot_general(sel, p, (((0,), (1,)), ((), ())),
                         precision=_HIGH, preferred_element_type=F32)
    ext_ref[...] = jnp.exp(lt)


def _tc2(gxl, gxr, edge_attr, We, att, off):
    cnt, d = gxl.shape
    eb = 2560
    ob = off // eb
    grid = (cnt // eb,)
    row = pl.BlockSpec((eb, d), lambda i: (i, 0))
    erow = pl.BlockSpec((eb, d), lambda i: (i + ob, 0))
    return pl.pallas_call(
        _tc2_body,
        grid=grid,
        in_specs=[row, row, erow,
                  pl.BlockSpec((d, d), lambda i: (0, 0)),
                  pl.BlockSpec((1, d), lambda i: (0, 0))],
        out_specs=[row, pl.BlockSpec((4, eb), lambda i: (0, i))],
        out_shape=[jax.ShapeDtypeStruct((cnt, d), F32),
                   jax.ShapeDtypeStruct((4, cnt), F32)],
    )(gxl, gxr, edge_attr, We, att.reshape(1, d))


def _tc3_body(*refs):
    nparts = (len(refs) - 3) // 2
    ow_refs = refs[:nparts]
    os_refs = refs[nparts:2 * nparts]
    x_ref, bconv_ref, out_ref = refs[2 * nparts:]
    ow = ow_refs[0][...]
    os_ = os_refs[0][...]
    for r in ow_refs[1:]:
        ow = ow + r[...]
    for r in os_refs[1:]:
        os_ = os_ + r[...]
    out_ref[...] = x_ref[...] + ow / (os_ + 1e-16) + bconv_ref[...]


def _tc3(ows, oss, x, bconv):
    n, d = x.shape
    nb = 2000
    grid = (n // nb,)
    row = pl.BlockSpec((nb, d), lambda i: (i, 0))
    nparts = len(ows)
    return pl.pallas_call(
        _tc3_body,
        grid=grid,
        in_specs=[row] * (2 * nparts + 1)
        + [pl.BlockSpec((1, d), lambda i: (0, 0))],
        out_specs=row,
        out_shape=jax.ShapeDtypeStruct((n, d), F32),
    )(*ows, *oss, x, bconv.reshape(1, d))


def _tc4_body(gs_ref, gd_ref, ea_ref, g2_ref, b2_ref, wm_ref, bm_ref,
              out_ref):
    d = 128
    gs = gs_ref[...]
    gd = gd_ref[...]
    mu = (jnp.sum(gs, -1, keepdims=True)
          + jnp.sum(gd, -1, keepdims=True)) / (2.0 * d)
    ds_ = gs - mu
    dd = gd - mu
    var = (jnp.sum(ds_ * ds_, -1, keepdims=True)
           + jnp.sum(dd * dd, -1, keepdims=True)) / (2.0 * d)
    rs = lax.rsqrt(var + 1e-5)
    a_s = jnp.maximum(ds_ * rs * g2_ref[:, :d] + b2_ref[:, :d], 0.0)
    a_d = jnp.maximum(dd * rs * g2_ref[:, d:] + b2_ref[:, d:], 0.0)
    eupd = (jnp.dot(a_s, wm_ref[0], precision=_HIGH,
                    preferred_element_type=F32)
            + jnp.dot(a_d, wm_ref[1], precision=_HIGH,
                      preferred_element_type=F32) + bm_ref[...])
    out_ref[...] = ea_ref[...] + eupd


def _tc4(gs, gd, edge_attr, ln2_g, ln2_b, Wm, bm, off, prev):
    e, d = edge_attr.shape
    cnt = gs.shape[0]
    eb = 2560
    ob = off // eb
    grid = (cnt // eb,)
    row = pl.BlockSpec((eb, d), lambda i: (i, 0))
    erow = pl.BlockSpec((eb, d), lambda i: (i + ob, 0))
    args = [gs, gd, edge_attr, ln2_g.reshape(1, 2 * d),
            ln2_b.reshape(1, 2 * d), Wm.reshape(2, d, d), bm.reshape(1, d)]
    in_specs = [row, row, erow,
                pl.BlockSpec((1, 2 * d), lambda i: (0, 0)),
                pl.BlockSpec((1, 2 * d), lambda i: (0, 0)),
                pl.BlockSpec((2, d, d), lambda i: (0, 0, 0)),
                pl.BlockSpec((1, d), lambda i: (0, 0))]
    kwargs = {}
    body = _tc4_body
    if prev is not None:
        args.append(prev)
        in_specs.append(pl.BlockSpec((8, d), lambda i: (0, 0)))
        kwargs["input_output_aliases"] = {7: 0}

        def body(gs_r, gd_r, ea_r, g2_r, b2_r, wm_r, bm_r, prev_r, out_r):
            del prev_r
            _tc4_body(gs_r, gd_r, ea_r, g2_r, b2_r, wm_r, bm_r, out_r)

    return pl.pallas_call(
        body,
        grid=grid,
        in_specs=in_specs,
        out_specs=erow,
        out_shape=jax.ShapeDtypeStruct((e, d), F32),
        **kwargs,
    )(*args)


def _sc_mesh():
    return plsc.VectorSubcoreMesh(core_axis_name="c", subcore_axis_name="s",
                                  num_cores=2)


def _sc_dual_gather(tbl_a, idx_a, tbl_b, idx_b, off, cnt):
    d = tbl_a.shape[1]
    pw = cnt // NW
    niter = pw // GK

    @functools.partial(
        pl.kernel, mesh=_sc_mesh(),
        out_type=[jax.ShapeDtypeStruct((cnt, d), F32),
                  jax.ShapeDtypeStruct((cnt, d), F32)],
        scratch_types=[pltpu.VMEM((GK,), jnp.int32),
                       pltpu.VMEM((GK, d), F32),
                       pltpu.VMEM((GK,), jnp.int32),
                       pltpu.VMEM((GK, d), F32),
                       pltpu.SemaphoreType.DMA,
                       pltpu.SemaphoreType.DMA],
    )
    def gather_k(ta_h, ia_h, tb_h, ib_h, oa_h, ob_h,
                 ia_v, ra_v, ib_v, rb_v, sem_a, sem_b):
        wid = lax.axis_index("s") * 2 + lax.axis_index("c")
        base0 = wid * pw

        def body(j, carry):
            base = base0 + j * GK
            pltpu.sync_copy(ia_h.at[pl.ds(off + base, GK)], ia_v)
            pltpu.sync_copy(ib_h.at[pl.ds(off + base, GK)], ib_v)
            cp_a = pltpu.async_copy(ta_h.at[ia_v], ra_v, sem_a)
            cp_b = pltpu.async_copy(tb_h.at[ib_v], rb_v, sem_b)
            cp_a.wait()
            cp_b.wait()
            pltpu.sync_copy(ra_v, oa_h.at[pl.ds(base, GK)])
            pltpu.sync_copy(rb_v, ob_h.at[pl.ds(base, GK)])
            return carry

        lax.fori_loop(0, niter, body, 0)

    return gather_k(tbl_a, idx_a, tbl_b, idx_b)


def _sc_scatter_add(w_rows, ex_flat, idx, zeros, n, off):
    d = 128
    cnt = w_rows.shape[0]
    pw = cnt // 16
    niter = pw // GK

    @functools.partial(
        pl.kernel, mesh=_sc_mesh(),
        out_type=[jax.ShapeDtypeStruct((n, d), F32),
                  jax.ShapeDtypeStruct((n, d), F32)],
        scratch_types=[pltpu.VMEM((GK,), jnp.int32),
                       pltpu.VMEM((GK, d), F32),
                       pltpu.VMEM((4, GK), F32),
                       pltpu.VMEM_SHARED((n, d), F32)],
    )
    def scatter_k(w_h, ex_h, idx_h, zeros_h, ow_h, os_h,
                  idx_v, rows_v, ex_v, acc_sh):
        cid = lax.axis_index("c")
        sid = lax.axis_index("s")

        @pl.when(sid == 0)
        def _():
            pltpu.sync_copy(zeros_h, acc_sh)

        plsc.subcore_barrier()
        lanes = lax.iota(jnp.int32, 16)

        @pl.when(cid == 0)
        def _():
            def body(j, carry):
                base = sid * pw + j * GK
                pltpu.sync_copy(idx_h.at[pl.ds(off + base, GK)], idx_v)
                pltpu.sync_copy(w_h.at[pl.ds(base, GK)], rows_v)
                pltpu.sync_copy(rows_v, acc_sh.at[idx_v], add=True)
                return carry

            lax.fori_loop(0, niter, body, 0)

        @pl.when(cid == 1)
        def _():
            def body(j, carry):
                base = sid * pw + j * GK
                pltpu.sync_copy(idx_h.at[pl.ds(off + base, GK)], idx_v)
                for h in range(4):
                    pltpu.sync_copy(ex_h.at[pl.ds(h * cnt + base, GK)],
                                    ex_v.at[h])

                def build(k, c2):
                    vh = [ex_v[h, pl.ds(k * 16, 16)] for h in range(4)]
                    for jj in range(16):
                        i = k * 16 + jj
                        for h in range(4):
                            sp = jnp.full((16,), vh[h][jj], F32)
                            rows_v[i, pl.ds(h * 32, 16)] = sp
                            rows_v[i, pl.ds(h * 32 + 16, 16)] = sp
                    return c2

                lax.fori_loop(0, GK // 16, build, 0)
                pltpu.sync_copy(rows_v, acc_sh.at[idx_v], add=True)
                return carry

            lax.fori_loop(0, niter, body, 0)

        plsc.subcore_barrier()

        @pl.when((sid == 0) & (cid == 0))
        def _():
            pltpu.sync_copy(acc_sh, ow_h)

        @pl.when((sid == 0) & (cid == 1))
        def _():
            pltpu.sync_copy(acc_sh, os_h)

    return scatter_k(w_rows, ex_flat, idx, zeros)


def kernel(x, edge_index, edge_attr, ln1_g, ln1_b, Wl, bl, Wr, br, We, att,
           bconv, ln2_g, ln2_b, Wm, bm):
    n, d = x.shape
    src = edge_index[0]
    dst = edge_index[1]

    e = edge_attr.shape[0]
    nblk = e // 2560
    nsl = 4
    per = nblk // nsl
    sizes = [per * 2560] * (nsl - 1) + [(nblk - per * (nsl - 1)) * 2560]
    offs = [sum(sizes[:i]) for i in range(nsl)]
    slices = list(zip(offs, sizes))

    xl, xr = _tc1(x, ln1_g, ln1_b, Wl, bl, Wr, br)
    zeros = jnp.zeros((n, d), F32)
    ows, oss = [], []
    for off, cnt in slices:
        gxl, gxr = _sc_dual_gather(xl, src, xr, dst, off, cnt)
        w_rows, ex_t = _tc2(gxl, gxr, edge_attr, We, att, off)
        ow, os_ = _sc_scatter_add(w_rows, ex_t.reshape(-1), dst, zeros, n, off)
        ows.append(ow)
        oss.append(os_)
    x_new = _tc3(ows, oss, x, bconv)
    out = None
    for off, cnt in slices:
        gs, gd = _sc_dual_gather(x_new, src, x_new, dst, off, cnt)
        out = _tc4(gs, gd, edge_attr, ln2_g, ln2_b, Wm, bm, off, out)
    return (x_new, out)


# speedup vs baseline: 1.2593x; 1.2593x over previous
import functools

import jax
import jax.numpy as jnp
from jax import lax
from jax.experimental import pallas as pl
from jax.experimental.pallas import tpu as pltpu
from jax.experimental.pallas import tpu_sc as plsc

F32 = jnp.float32
_HIGH = lax.Precision.HIGHEST

NW = 32
GK = 80


def _tc1_body(x_ref, g_ref, b_ref, wl_ref, bl_ref, wr_ref, br_ref,
              xl_ref, xr_ref):
    xv = x_ref[...]
    mu = jnp.mean(xv, axis=-1, keepdims=True)
    var = jnp.mean((xv - mu) * (xv - mu), axis=-1, keepdims=True)
    h = (xv - mu) * lax.rsqrt(var + 1e-5) * g_ref[...] + b_ref[...]
    h = jnp.maximum(h, 0.0)
    xl_ref[...] = jnp.dot(h, wl_ref[...], precision=_HIGH,
                          preferred_element_type=F32) + bl_ref[...]
    xr_ref[...] = jnp.dot(h, wr_ref[...], precision=_HIGH,
                          preferred_element_type=F32) + br_ref[...]


def _tc1(x, ln1_g, ln1_b, Wl, bl, Wr, br):
    n, d = x.shape
    nb = 2000
    grid = (n // nb,)
    row = pl.BlockSpec((nb, d), lambda i: (i, 0))
    full = pl.BlockSpec((1, d), lambda i: (0, 0))
    w = pl.BlockSpec((d, d), lambda i: (0, 0))
    return pl.pallas_call(
        _tc1_body,
        grid=grid,
        in_specs=[row, full, full, w, full, w, full],
        out_specs=[row, row],
        out_shape=[jax.ShapeDtypeStruct((n, d), F32)] * 2,
    )(x, ln1_g.reshape(1, d), ln1_b.reshape(1, d), Wl, bl.reshape(1, d),
      Wr, br.reshape(1, d))


def _tc2_body(gxl_ref, gxr_ref, ea_ref, we_ref, att_ref, out_ref, ext_ref):
    d = 128
    ea = jnp.dot(ea_ref[...], we_ref[...], precision=_HIGH,
                 preferred_element_type=F32)
    e = gxl_ref[...] + gxr_ref[...] + ea
    e = jnp.where(e >= 0, e, 0.2 * e)
    p = e * att_ref[...]
    r128 = lax.broadcasted_iota(jnp.int32, (d, 4), 0)
    c4 = lax.broadcasted_iota(jnp.int32, (d, 4), 1)
    sel = (r128 // 32 == c4).astype(F32)
    logits = jnp.dot(p, sel, precision=_HIGH, preferred_element_type=F32)
    ex = jnp.exp(logits)
    r4 = lax.broadcasted_iota(jnp.int32, (4, d), 0)
    c128 = lax.broadcasted_iota(jnp.int32, (4, d), 1)
    bsel = (c128 // 32 == r4).astype(F32)
    exb = jnp.dot(ex, bsel, precision=_HIGH, preferred_element_type=F32)
    out_ref[...] = gxl_ref[...] * exb
    ext_ref[...] = exb


def _tc2(gxl, gxr, edge_attr, We, att, off):
    cnt, d = gxl.shape
    eb = 2560
    ob = off // eb
    grid = (cnt // eb,)
    row = pl.BlockSpec((eb, d), lambda i: (i, 0))
    erow = pl.BlockSpec((eb, d), lambda i: (i + ob, 0))
    return pl.pallas_call(
        _tc2_body,
        grid=grid,
        in_specs=[row, row, erow,
                  pl.BlockSpec((d, d), lambda i: (0, 0)),
                  pl.BlockSpec((1, d), lambda i: (0, 0))],
        out_specs=[row, row],
        out_shape=[jax.ShapeDtypeStruct((cnt, d), F32),
                   jax.ShapeDtypeStruct((cnt, d), F32)],
    )(gxl, gxr, edge_attr, We, att.reshape(1, d))


def _tc3_body(*refs):
    nparts = (len(refs) - 3) // 2
    ow_refs = refs[:nparts]
    os_refs = refs[nparts:2 * nparts]
    x_ref, bconv_ref, out_ref = refs[2 * nparts:]
    ow = ow_refs[0][...]
    os_ = os_refs[0][...]
    for r in ow_refs[1:]:
        ow = ow + r[...]
    for r in os_refs[1:]:
        os_ = os_ + r[...]
    out_ref[...] = x_ref[...] + ow / (os_ + 1e-16) + bconv_ref[...]


def _tc3(ows, oss, x, bconv):
    n, d = x.shape
    nb = 2000
    grid = (n // nb,)
    row = pl.BlockSpec((nb, d), lambda i: (i, 0))
    nparts = len(ows)
    return pl.pallas_call(
        _tc3_body,
        grid=grid,
        in_specs=[row] * (2 * nparts + 1)
        + [pl.BlockSpec((1, d), lambda i: (0, 0))],
        out_specs=row,
        out_shape=jax.ShapeDtypeStruct((n, d), F32),
    )(*ows, *oss, x, bconv.reshape(1, d))


def _tc4_body(gs_ref, gd_ref, ea_ref, g2_ref, b2_ref, wm_ref, bm_ref,
              out_ref):
    d = 128
    gs = gs_ref[...]
    gd = gd_ref[...]
    mu = (jnp.sum(gs, -1, keepdims=True)
          + jnp.sum(gd, -1, keepdims=True)) / (2.0 * d)
    ds_ = gs - mu
    dd = gd - mu
    var = (jnp.sum(ds_ * ds_, -1, keepdims=True)
           + jnp.sum(dd * dd, -1, keepdims=True)) / (2.0 * d)
    rs = lax.rsqrt(var + 1e-5)
    a_s = jnp.maximum(ds_ * rs * g2_ref[:, :d] + b2_ref[:, :d], 0.0)
    a_d = jnp.maximum(dd * rs * g2_ref[:, d:] + b2_ref[:, d:], 0.0)
    eupd = (jnp.dot(a_s, wm_ref[0], precision=_HIGH,
                    preferred_element_type=F32)
            + jnp.dot(a_d, wm_ref[1], precision=_HIGH,
                      preferred_element_type=F32) + bm_ref[...])
    out_ref[...] = ea_ref[...] + eupd


def _tc4(gs, gd, edge_attr, ln2_g, ln2_b, Wm, bm, off, prev):
    e, d = edge_attr.shape
    cnt = gs.shape[0]
    eb = 2560
    ob = off // eb
    grid = (cnt // eb,)
    row = pl.BlockSpec((eb, d), lambda i: (i, 0))
    erow = pl.BlockSpec((eb, d), lambda i: (i + ob, 0))
    args = [gs, gd, edge_attr, ln2_g.reshape(1, 2 * d),
            ln2_b.reshape(1, 2 * d), Wm.reshape(2, d, d), bm.reshape(1, d)]
    in_specs = [row, row, erow,
                pl.BlockSpec((1, 2 * d), lambda i: (0, 0)),
                pl.BlockSpec((1, 2 * d), lambda i: (0, 0)),
                pl.BlockSpec((2, d, d), lambda i: (0, 0, 0)),
                pl.BlockSpec((1, d), lambda i: (0, 0))]
    kwargs = {}
    body = _tc4_body
    if prev is not None:
        args.append(prev)
        in_specs.append(pl.BlockSpec((8, d), lambda i: (0, 0)))
        kwargs["input_output_aliases"] = {7: 0}

        def body(gs_r, gd_r, ea_r, g2_r, b2_r, wm_r, bm_r, prev_r, out_r):
            del prev_r
            _tc4_body(gs_r, gd_r, ea_r, g2_r, b2_r, wm_r, bm_r, out_r)

    return pl.pallas_call(
        body,
        grid=grid,
        in_specs=in_specs,
        out_specs=erow,
        out_shape=jax.ShapeDtypeStruct((e, d), F32),
        **kwargs,
    )(*args)


def _sc_mesh():
    return plsc.VectorSubcoreMesh(core_axis_name="c", subcore_axis_name="s",
                                  num_cores=2)


def _sc_dual_gather(tbl_a, idx_a, tbl_b, idx_b, off, cnt):
    d = tbl_a.shape[1]
    pw = cnt // NW
    niter = pw // GK

    @functools.partial(
        pl.kernel, mesh=_sc_mesh(),
        out_type=[jax.ShapeDtypeStruct((cnt, d), F32),
                  jax.ShapeDtypeStruct((cnt, d), F32)],
        scratch_types=[pltpu.VMEM((2, GK), jnp.int32),
                       pltpu.VMEM((2, GK, d), F32),
                       pltpu.VMEM((2, GK), jnp.int32),
                       pltpu.VMEM((2, GK, d), F32),
                       pltpu.SemaphoreType.DMA,
                       pltpu.SemaphoreType.DMA,
                       pltpu.SemaphoreType.DMA,
                       pltpu.SemaphoreType.DMA],
    )
    def gather_k(ta_h, ia_h, tb_h, ib_h, oa_h, ob_h,
                 ia_v, ra_v, ib_v, rb_v, sa0, sa1, sb0, sb1):
        wid = lax.axis_index("s") * 2 + lax.axis_index("c")
        base0 = wid * pw
        sems = [(sa0, sb0), (sa1, sb1)]

        def start(j, b):
            sa, sb = sems[b]
            pltpu.sync_copy(ia_h.at[pl.ds(off + base0 + j * GK, GK)],
                            ia_v.at[b])
            pltpu.sync_copy(ib_h.at[pl.ds(off + base0 + j * GK, GK)],
                            ib_v.at[b])
            pltpu.async_copy(ta_h.at[ia_v.at[b]], ra_v.at[b], sa)
            pltpu.async_copy(tb_h.at[ib_v.at[b]], rb_v.at[b], sb)

        def finish(j, b):
            sa, sb = sems[b]
            pltpu.make_async_copy(ta_h.at[ia_v.at[b]], ra_v.at[b], sa).wait()
            pltpu.make_async_copy(tb_h.at[ib_v.at[b]], rb_v.at[b], sb).wait()
            pltpu.sync_copy(ra_v.at[b], oa_h.at[pl.ds(base0 + j * GK, GK)])
            pltpu.sync_copy(rb_v.at[b], ob_h.at[pl.ds(base0 + j * GK, GK)])

        start(0, 0)

        def pair(p, carry):
            j0 = 2 * p
            start(j0 + 1, 1)
            finish(j0, 0)

            @pl.when(j0 + 2 < niter)
            def _():
                start(j0 + 2, 0)

            finish(j0 + 1, 1)
            return carry

        lax.fori_loop(0, niter // 2, pair, 0)
        if niter % 2:
            finish(niter - 1, 0)

    return gather_k(tbl_a, idx_a, tbl_b, idx_b)


def _sc_scatter_add(w_rows, exb_rows, idx, zeros, n, off):
    d = 128
    cnt = w_rows.shape[0]
    pw = cnt // 16
    niter = pw // GK

    @functools.partial(
        pl.kernel, mesh=_sc_mesh(),
        out_type=[jax.ShapeDtypeStruct((n, d), F32),
                  jax.ShapeDtypeStruct((n, d), F32)],
        scratch_types=[pltpu.VMEM((GK,), jnp.int32),
                       pltpu.VMEM((GK, d), F32),
                       pltpu.VMEM_SHARED((n, d), F32)],
    )
    def scatter_k(w_h, ex_h, idx_h, zeros_h, ow_h, os_h,
                  idx_v, rows_v, acc_sh):
        cid = lax.axis_index("c")
        sid = lax.axis_index("s")

        @pl.when(sid == 0)
        def _():
            pltpu.sync_copy(zeros_h, acc_sh)

        plsc.subcore_barrier()

        @pl.when(cid == 0)
        def _():
            def body(j, carry):
                base = sid * pw + j * GK
                pltpu.sync_copy(idx_h.at[pl.ds(off + base, GK)], idx_v)
                pltpu.sync_copy(w_h.at[pl.ds(base, GK)], rows_v)
                pltpu.sync_copy(rows_v, acc_sh.at[idx_v], add=True)
                return carry

            lax.fori_loop(0, niter, body, 0)

        @pl.when(cid == 1)
        def _():
            def body(j, carry):
                base = sid * pw + j * GK
                pltpu.sync_copy(idx_h.at[pl.ds(off + base, GK)], idx_v)
                pltpu.sync_copy(ex_h.at[pl.ds(base, GK)], rows_v)
                pltpu.sync_copy(rows_v, acc_sh.at[idx_v], add=True)
                return carry

            lax.fori_loop(0, niter, body, 0)

        plsc.subcore_barrier()

        @pl.when((sid == 0) & (cid == 0))
        def _():
            pltpu.sync_copy(acc_sh, ow_h)

        @pl.when((sid == 0) & (cid == 1))
        def _():
            pltpu.sync_copy(acc_sh, os_h)

    return scatter_k(w_rows, exb_rows, idx, zeros)


def kernel(x, edge_index, edge_attr, ln1_g, ln1_b, Wl, bl, Wr, br, We, att,
           bconv, ln2_g, ln2_b, Wm, bm):
    n, d = x.shape
    src = edge_index[0]
    dst = edge_index[1]

    e = edge_attr.shape[0]
    nblk = e // 2560
    nsl = 4
    per = nblk // nsl
    sizes = [per * 2560] * (nsl - 1) + [(nblk - per * (nsl - 1)) * 2560]
    offs = [sum(sizes[:i]) for i in range(nsl)]
    slices = list(zip(offs, sizes))

    xl, xr = _tc1(x, ln1_g, ln1_b, Wl, bl, Wr, br)
    zeros = jnp.zeros((n, d), F32)
    ows, oss = [], []
    for off, cnt in slices:
        gxl, gxr = _sc_dual_gather(xl, src, xr, dst, off, cnt)
        w_rows, exb_rows = _tc2(gxl, gxr, edge_attr, We, att, off)
        ow, os_ = _sc_scatter_add(w_rows, exb_rows, dst, zeros, n, off)
        ows.append(ow)
        oss.append(os_)
    x_new = _tc3(ows, oss, x, bconv)
    out = None
    for off, cnt in slices:
        gs, gd = _sc_dual_gather(x_new, src, x_new, dst, off, cnt)
        out = _tc4(gs, gd, edge_attr, ln2_g, ln2_b, Wm, bm, off, out)
    return (x_new, out)


# speedup vs baseline: 1.3898x; 1.1037x over previous
import functools

import jax
import jax.numpy as jnp
from jax import lax
from jax.experimental import pallas as pl
from jax.experimental.pallas import tpu as pltpu
from jax.experimental.pallas import tpu_sc as plsc

F32 = jnp.float32
_HIGH = lax.Precision.HIGHEST

NW = 32
GK = 80


def _tc1_body(x_ref, g_ref, b_ref, wl_ref, bl_ref, wr_ref, br_ref,
              xl_ref, xr_ref):
    xv = x_ref[...]
    mu = jnp.mean(xv, axis=-1, keepdims=True)
    var = jnp.mean((xv - mu) * (xv - mu), axis=-1, keepdims=True)
    h = (xv - mu) * lax.rsqrt(var + 1e-5) * g_ref[...] + b_ref[...]
    h = jnp.maximum(h, 0.0)
    xl_ref[...] = jnp.dot(h, wl_ref[...], precision=_HIGH,
                          preferred_element_type=F32) + bl_ref[...]
    xr_ref[...] = jnp.dot(h, wr_ref[...], precision=_HIGH,
                          preferred_element_type=F32) + br_ref[...]


def _tc1(x, ln1_g, ln1_b, Wl, bl, Wr, br):
    n, d = x.shape
    nb = 2000
    grid = (n // nb,)
    row = pl.BlockSpec((nb, d), lambda i: (i, 0))
    full = pl.BlockSpec((1, d), lambda i: (0, 0))
    w = pl.BlockSpec((d, d), lambda i: (0, 0))
    return pl.pallas_call(
        _tc1_body,
        grid=grid,
        in_specs=[row, full, full, w, full, w, full],
        out_specs=[row, row],
        out_shape=[jax.ShapeDtypeStruct((n, d), F32)] * 2,
    )(x, ln1_g.reshape(1, d), ln1_b.reshape(1, d), Wl, bl.reshape(1, d),
      Wr, br.reshape(1, d))


def _tc2_body(gxl_ref, gxr_ref, ea_ref, we_ref, att_ref, out_ref, ext_ref):
    d = 128
    ea = jnp.dot(ea_ref[...], we_ref[...], precision=_HIGH,
                 preferred_element_type=F32)
    e = gxl_ref[...] + gxr_ref[...] + ea
    e = jnp.where(e >= 0, e, 0.2 * e)
    p = e * att_ref[...]
    r128 = lax.broadcasted_iota(jnp.int32, (d, 4), 0)
    c4 = lax.broadcasted_iota(jnp.int32, (d, 4), 1)
    sel = (r128 // 32 == c4).astype(F32)
    logits = jnp.dot(p, sel, precision=_HIGH, preferred_element_type=F32)
    ex = jnp.exp(logits)
    r4 = lax.broadcasted_iota(jnp.int32, (4, d), 0)
    c128 = lax.broadcasted_iota(jnp.int32, (4, d), 1)
    bsel = (c128 // 32 == r4).astype(F32)
    exb = jnp.dot(ex, bsel, precision=_HIGH, preferred_element_type=F32)
    out_ref[...] = gxl_ref[...] * exb
    ext_ref[...] = exb


def _tc2(gxl, gxr, edge_attr, We, att, off):
    cnt, d = gxl.shape
    eb = 2560
    ob = off // eb
    grid = (cnt // eb,)
    row = pl.BlockSpec((eb, d), lambda i: (i, 0))
    erow = pl.BlockSpec((eb, d), lambda i: (i + ob, 0))
    return pl.pallas_call(
        _tc2_body,
        grid=grid,
        in_specs=[row, row, erow,
                  pl.BlockSpec((d, d), lambda i: (0, 0)),
                  pl.BlockSpec((1, d), lambda i: (0, 0))],
        out_specs=[row, row],
        out_shape=[jax.ShapeDtypeStruct((cnt, d), F32),
                   jax.ShapeDtypeStruct((cnt, d), F32)],
    )(gxl, gxr, edge_attr, We, att.reshape(1, d))


def _tc3_body(*refs):
    nparts = (len(refs) - 3) // 2
    ow_refs = refs[:nparts]
    os_refs = refs[nparts:2 * nparts]
    x_ref, bconv_ref, out_ref = refs[2 * nparts:]
    ow = ow_refs[0][...]
    os_ = os_refs[0][...]
    for r in ow_refs[1:]:
        ow = ow + r[...]
    for r in os_refs[1:]:
        os_ = os_ + r[...]
    out_ref[...] = x_ref[...] + ow / (os_ + 1e-16) + bconv_ref[...]


def _tc3(ows, oss, x, bconv):
    n, d = x.shape
    nb = 2000
    grid = (n // nb,)
    row = pl.BlockSpec((nb, d), lambda i: (i, 0))
    nparts = len(ows)
    return pl.pallas_call(
        _tc3_body,
        grid=grid,
        in_specs=[row] * (2 * nparts + 1)
        + [pl.BlockSpec((1, d), lambda i: (0, 0))],
        out_specs=row,
        out_shape=jax.ShapeDtypeStruct((n, d), F32),
    )(*ows, *oss, x, bconv.reshape(1, d))


def _tc4_body(gs_ref, gd_ref, ea_ref, g2_ref, b2_ref, wm_ref, bm_ref,
              out_ref):
    d = 128
    gs = gs_ref[...]
    gd = gd_ref[...]
    mu = (jnp.sum(gs, -1, keepdims=True)
          + jnp.sum(gd, -1, keepdims=True)) / (2.0 * d)
    ds_ = gs - mu
    dd = gd - mu
    var = (jnp.sum(ds_ * ds_, -1, keepdims=True)
           + jnp.sum(dd * dd, -1, keepdims=True)) / (2.0 * d)
    rs = lax.rsqrt(var + 1e-5)
    a_s = jnp.maximum(ds_ * rs * g2_ref[:, :d] + b2_ref[:, :d], 0.0)
    a_d = jnp.maximum(dd * rs * g2_ref[:, d:] + b2_ref[:, d:], 0.0)
    eupd = (jnp.dot(a_s, wm_ref[0], precision=_HIGH,
                    preferred_element_type=F32)
            + jnp.dot(a_d, wm_ref[1], precision=_HIGH,
                      preferred_element_type=F32) + bm_ref[...])
    out_ref[...] = ea_ref[...] + eupd


def _tc4(gs, gd, edge_attr, ln2_g, ln2_b, Wm, bm, off, prev):
    e, d = edge_attr.shape
    cnt = gs.shape[0]
    eb = 2560
    ob = off // eb
    grid = (cnt // eb,)
    row = pl.BlockSpec((eb, d), lambda i: (i, 0))
    erow = pl.BlockSpec((eb, d), lambda i: (i + ob, 0))
    args = [gs, gd, edge_attr, ln2_g.reshape(1, 2 * d),
            ln2_b.reshape(1, 2 * d), Wm.reshape(2, d, d), bm.reshape(1, d)]
    in_specs = [row, row, erow,
                pl.BlockSpec((1, 2 * d), lambda i: (0, 0)),
                pl.BlockSpec((1, 2 * d), lambda i: (0, 0)),
                pl.BlockSpec((2, d, d), lambda i: (0, 0, 0)),
                pl.BlockSpec((1, d), lambda i: (0, 0))]
    kwargs = {}
    body = _tc4_body
    if prev is not None:
        args.append(prev)
        in_specs.append(pl.BlockSpec((8, d), lambda i: (0, 0)))
        kwargs["input_output_aliases"] = {7: 0}

        def body(gs_r, gd_r, ea_r, g2_r, b2_r, wm_r, bm_r, prev_r, out_r):
            del prev_r
            _tc4_body(gs_r, gd_r, ea_r, g2_r, b2_r, wm_r, bm_r, out_r)

    return pl.pallas_call(
        body,
        grid=grid,
        in_specs=in_specs,
        out_specs=erow,
        out_shape=jax.ShapeDtypeStruct((e, d), F32),
        **kwargs,
    )(*args)


def _sc_mesh():
    return plsc.VectorSubcoreMesh(core_axis_name="c", subcore_axis_name="s",
                                  num_cores=2)


def _sc_dual_gather(tbl_a, idx_a, tbl_b, idx_b, off, cnt):
    d = tbl_a.shape[1]
    pw = cnt // NW
    niter = pw // GK

    @functools.partial(
        pl.kernel, mesh=_sc_mesh(),
        out_type=[jax.ShapeDtypeStruct((cnt, d), F32),
                  jax.ShapeDtypeStruct((cnt, d), F32)],
        scratch_types=[pltpu.VMEM((2, GK), jnp.int32),
                       pltpu.VMEM((2, GK, d), F32),
                       pltpu.VMEM((2, GK), jnp.int32),
                       pltpu.VMEM((2, GK, d), F32),
                       pltpu.SemaphoreType.DMA,
                       pltpu.SemaphoreType.DMA,
                       pltpu.SemaphoreType.DMA,
                       pltpu.SemaphoreType.DMA],
    )
    def gather_k(ta_h, ia_h, tb_h, ib_h, oa_h, ob_h,
                 ia_v, ra_v, ib_v, rb_v, sa0, sa1, sb0, sb1):
        wid = lax.axis_index("s") * 2 + lax.axis_index("c")
        base0 = wid * pw
        sems = [(sa0, sb0), (sa1, sb1)]

        def start(j, b):
            sa, sb = sems[b]
            pltpu.sync_copy(ia_h.at[pl.ds(off + base0 + j * GK, GK)],
                            ia_v.at[b])
            pltpu.sync_copy(ib_h.at[pl.ds(off + base0 + j * GK, GK)],
                            ib_v.at[b])
            pltpu.async_copy(ta_h.at[ia_v.at[b]], ra_v.at[b], sa)
            pltpu.async_copy(tb_h.at[ib_v.at[b]], rb_v.at[b], sb)

        def finish(j, b):
            sa, sb = sems[b]
            pltpu.make_async_copy(ta_h.at[ia_v.at[b]], ra_v.at[b], sa).wait()
            pltpu.make_async_copy(tb_h.at[ib_v.at[b]], rb_v.at[b], sb).wait()
            pltpu.sync_copy(ra_v.at[b], oa_h.at[pl.ds(base0 + j * GK, GK)])
            pltpu.sync_copy(rb_v.at[b], ob_h.at[pl.ds(base0 + j * GK, GK)])

        start(0, 0)

        def pair(p, carry):
            j0 = 2 * p
            start(j0 + 1, 1)
            finish(j0, 0)

            @pl.when(j0 + 2 < niter)
            def _():
                start(j0 + 2, 0)

            finish(j0 + 1, 1)
            return carry

        lax.fori_loop(0, niter // 2, pair, 0)
        if niter % 2:
            finish(niter - 1, 0)

    return gather_k(tbl_a, idx_a, tbl_b, idx_b)


def _sc_scatter_add(w_rows, exb_rows, idx, zeros, n, off):
    d = 128
    cnt = w_rows.shape[0]
    pw = cnt // 16
    niter = pw // GK

    @functools.partial(
        pl.kernel, mesh=_sc_mesh(),
        out_type=[jax.ShapeDtypeStruct((n, d), F32),
                  jax.ShapeDtypeStruct((n, d), F32)],
        scratch_types=[pltpu.VMEM((2, GK), jnp.int32),
                       pltpu.VMEM((2, GK, d), F32),
                       pltpu.VMEM_SHARED((n, d), F32),
                       pltpu.SemaphoreType.DMA,
                       pltpu.SemaphoreType.DMA,
                       pltpu.SemaphoreType.DMA,
                       pltpu.SemaphoreType.DMA],
    )
    def scatter_k(w_h, ex_h, idx_h, zeros_h, ow_h, os_h,
                  idx_v, rows_v, acc_sh, si0, si1, sr0, sr1):
        cid = lax.axis_index("c")
        sid = lax.axis_index("s")

        @pl.when(sid == 0)
        def _():
            pltpu.sync_copy(zeros_h, acc_sh)

        plsc.subcore_barrier()
        sems = [(si0, sr0), (si1, sr1)]

        def run(src_h):
            def start(j, b):
                si, sr = sems[b]
                base = sid * pw + j * GK
                pltpu.async_copy(idx_h.at[pl.ds(off + base, GK)],
                                 idx_v.at[b], si)
                pltpu.async_copy(src_h.at[pl.ds(base, GK)], rows_v.at[b], sr)

            def finish(j, b):
                si, sr = sems[b]
                base = sid * pw + j * GK
                pltpu.make_async_copy(idx_h.at[pl.ds(off + base, GK)],
                                      idx_v.at[b], si).wait()
                pltpu.make_async_copy(src_h.at[pl.ds(base, GK)],
                                      rows_v.at[b], sr).wait()
                pltpu.sync_copy(rows_v.at[b], acc_sh.at[idx_v.at[b]],
                                add=True)

            start(0, 0)

            def pair(p, carry):
                j0 = 2 * p
                start(j0 + 1, 1)
                finish(j0, 0)

                @pl.when(j0 + 2 < niter)
                def _():
                    start(j0 + 2, 0)

                finish(j0 + 1, 1)
                return carry

            lax.fori_loop(0, niter // 2, pair, 0)
            if niter % 2:
                finish(niter - 1, 0)

        @pl.when(cid == 0)
        def _():
            run(w_h)

        @pl.when(cid == 1)
        def _():
            run(ex_h)

        plsc.subcore_barrier()

        @pl.when((sid == 0) & (cid == 0))
        def _():
            pltpu.sync_copy(acc_sh, ow_h)

        @pl.when((sid == 0) & (cid == 1))
        def _():
            pltpu.sync_copy(acc_sh, os_h)

    return scatter_k(w_rows, exb_rows, idx, zeros)


def kernel(x, edge_index, edge_attr, ln1_g, ln1_b, Wl, bl, Wr, br, We, att,
           bconv, ln2_g, ln2_b, Wm, bm):
    n, d = x.shape
    src = edge_index[0]
    dst = edge_index[1]

    e = edge_attr.shape[0]
    nblk = e // 2560
    nsl = 4
    per = nblk // nsl
    sizes = [per * 2560] * (nsl - 1) + [(nblk - per * (nsl - 1)) * 2560]
    offs = [sum(sizes[:i]) for i in range(nsl)]
    slices = list(zip(offs, sizes))

    xl, xr = _tc1(x, ln1_g, ln1_b, Wl, bl, Wr, br)
    zeros = jnp.zeros((n, d), F32)
    ows, oss = [], []
    for off, cnt in slices:
        gxl, gxr = _sc_dual_gather(xl, src, xr, dst, off, cnt)
        w_rows, exb_rows = _tc2(gxl, gxr, edge_attr, We, att, off)
        ow, os_ = _sc_scatter_add(w_rows, exb_rows, dst, zeros, n, off)
        ows.append(ow)
        oss.append(os_)
    x_new = _tc3(ows, oss, x, bconv)
    out = None
    for off, cnt in slices:
        gs, gd = _sc_dual_gather(x_new, src, x_new, dst, off, cnt)
        out = _tc4(gs, gd, edge_attr, ln2_g, ln2_b, Wm, bm, off, out)
    return (x_new, out)
